# baseline (speedup 1.0000x reference)
import jax
import jax.numpy as jnp
from jax.experimental import pallas as pl

N = 32768
D = 512
C = 10
S = 16
BN = 2048
GRID = N // BN
NSPLIT = 4
BNS = BN // NSPLIT


def _phase_a(*refs):
    emb_refs = refs[:NSPLIT]
    bi_ref, w_ref, b_ref, hact_ref, sums_ref = refs[NSPLIT:]
    b = b_ref[...]
    bi = bi_ref[...]
    part = jnp.zeros((S, 2 * C), jnp.float32)
    for k in range(NSPLIT):
        h = jnp.dot(emb_refs[k][...], w_ref[...],
                    preferred_element_type=jnp.float32) + b
        lane = jax.lax.broadcasted_iota(jnp.int32, (BNS, 2 * C), 1)
        hact = jnp.where(lane < C, jax.nn.softplus(h), h)
        hact_ref[k * BNS:(k + 1) * BNS, :] = hact
        oh = (bi[k * BNS:(k + 1) * BNS, :]
              == jax.lax.broadcasted_iota(jnp.int32, (BNS, S), 1)
              ).astype(jnp.float32)
        part = part + jax.lax.dot_general(
            oh, hact, (((0,), (0,)), ((), ())),
            preferred_element_type=jnp.float32)

    @pl.when(pl.program_id(0) == 0)
    def _init():
        sums_ref[...] = part

    @pl.when(pl.program_id(0) != 0)
    def _acc():
        sums_ref[...] += part


def _phase_b(hact_ref, bi_ref, sums_ref, qtot_ref, q_ref):
    sums = sums_ref[...]
    wsum = sums[:, :C]
    qsum = sums[:, C:]
    dq = qtot_ref[...] - qsum
    fsys = jnp.where(wsum > 0, dq / jnp.where(wsum > 0, wsum, 1.0), 0.0)
    bi = bi_ref[...]
    oh = (bi == jax.lax.broadcasted_iota(jnp.int32, (BN, S), 1)
          ).astype(jnp.float32)
    f = jnp.dot(oh, fsys, preferred_element_type=jnp.float32)
    hact = hact_ref[...]
    q_ref[...] = hact[:, C:] + hact[:, :C] * f


@jax.jit
def _run(embedding, batch_index, total_charge, W_wi, b_wi, W_qi, b_qi):
    bi2 = batch_index.reshape(N, 1)
    w_cat = jnp.concatenate([W_wi, W_qi], axis=1)
    b_cat = jnp.concatenate([b_wi, b_qi]).reshape(1, 2 * C)
    qtot = total_charge.reshape(S, 1)
    def _emb_spec(k):
        return pl.BlockSpec((BNS, D), lambda i, k=k: (i * NSPLIT + k, 0))

    hact, sums = pl.pallas_call(
        _phase_a,
        grid=(GRID,),
        in_specs=[_emb_spec(k) for k in range(NSPLIT)] + [
            pl.BlockSpec((BN, 1), lambda i: (i, 0)),
            pl.BlockSpec((D, 2 * C), lambda i: (0, 0)),
            pl.BlockSpec((1, 2 * C), lambda i: (0, 0)),
        ],
        out_specs=[
            pl.BlockSpec((BN, 2 * C), lambda i: (i, 0)),
            pl.BlockSpec((S, 2 * C), lambda i: (0, 0)),
        ],
        out_shape=[
            jax.ShapeDtypeStruct((N, 2 * C), jnp.float32),
            jax.ShapeDtypeStruct((S, 2 * C), jnp.float32),
        ],
    )(*([embedding] * NSPLIT), bi2, w_cat, b_cat)

    q = pl.pallas_call(
        _phase_b,
        grid=(GRID,),
        in_specs=[
            pl.BlockSpec((BN, 2 * C), lambda i: (i, 0)),
            pl.BlockSpec((BN, 1), lambda i: (i, 0)),
            pl.BlockSpec((S, 2 * C), lambda i: (0, 0)),
            pl.BlockSpec((S, 1), lambda i: (0, 0)),
        ],
        out_specs=pl.BlockSpec((BN, C), lambda i: (i, 0)),
        out_shape=jax.ShapeDtypeStruct((N, C), jnp.float32),
    )(hact, bi2, sums, qtot)
    return q


def kernel(embedding, coordinates, batch_index, natoms, total_charge,
           W_wi, b_wi, W_qi, b_qi):
    del coordinates, natoms
    return _run(embedding.astype(jnp.float32), batch_index,
                total_charge.astype(jnp.float32), W_wi, b_wi, W_qi, b_qi)


# speedup vs baseline: 2.7795x; 2.7795x over previous
import jax
import jax.numpy as jnp
from jax.experimental import pallas as pl
from jax.experimental.pallas import tpu as pltpu

N = 32768
D = 512
C = 10
S = 16
BN = 2048
GRID = N // BN


def _fused(emb_ref, bi_ref, w_ref, b_ref, qtot_ref, qt_ref,
           hact_s, oh_s, sums_s):
    i = pl.program_id(0)
    h = jax.lax.dot_general(
        w_ref[...], emb_ref[...], (((0,), (1,)), ((), ())),
        preferred_element_type=jnp.float32) + b_ref[...]
    row = jax.lax.broadcasted_iota(jnp.int32, (2 * C, BN), 0)
    hact = jnp.where(row < C, jax.nn.softplus(h), h)
    hact_s[:, pl.ds(i * BN, BN)] = hact

    bi = bi_ref[...]
    oh = (bi == jax.lax.broadcasted_iota(jnp.int32, (S, BN), 0)
          ).astype(jnp.float32)
    oh_s[:, pl.ds(i * BN, BN)] = oh
    part = jax.lax.dot_general(
        oh, hact, (((1,), (1,)), ((), ())),
        preferred_element_type=jnp.float32)

    @pl.when(i == 0)
    def _init():
        sums_s[...] = part

    @pl.when(i != 0)
    def _acc():
        sums_s[...] += part

    @pl.when(i == GRID - 1)
    def _finale():
        sums = sums_s[...]
        wsum = sums[:, :C]
        qsum = sums[:, C:]
        dq = qtot_ref[...] - qsum
        fsys = jnp.where(wsum > 0, dq / jnp.where(wsum > 0, wsum, 1.0), 0.0)
        f = jax.lax.dot_general(
            fsys, oh_s[...], (((0,), (0,)), ((), ())),
            preferred_element_type=jnp.float32)
        hall = hact_s[...]
        qt_ref[...] = hall[C:, :] + hall[:C, :] * f


@jax.jit
def _run(embedding, batch_index, total_charge, W_wi, b_wi, W_qi, b_qi):
    bi_row = batch_index.reshape(1, N)
    w_cat = jnp.concatenate([W_wi, W_qi], axis=1)
    b_cat = jnp.concatenate([b_wi, b_qi]).reshape(2 * C, 1)
    qtot = total_charge.reshape(S, 1)

    q_t = pl.pallas_call(
        _fused,
        grid=(GRID,),
        in_specs=[
            pl.BlockSpec((BN, D), lambda i: (i, 0)),
            pl.BlockSpec((1, BN), lambda i: (0, i)),
            pl.BlockSpec((D, 2 * C), lambda i: (0, 0)),
            pl.BlockSpec((2 * C, 1), lambda i: (0, 0)),
            pl.BlockSpec((S, 1), lambda i: (0, 0)),
        ],
        out_specs=pl.BlockSpec((C, N), lambda i: (0, 0)),
        out_shape=jax.ShapeDtypeStruct((C, N), jnp.float32),
        scratch_shapes=[
            pltpu.VMEM((2 * C, N), jnp.float32),
            pltpu.VMEM((S, N), jnp.float32),
            pltpu.VMEM((S, 2 * C), jnp.float32),
        ],
    )(embedding, bi_row, w_cat, b_cat, qtot)
    return q_t.T


def kernel(embedding, coordinates, batch_index, natoms, total_charge,
           W_wi, b_wi, W_qi, b_qi):
    del coordinates, natoms
    return _run(embedding.astype(jnp.float32), batch_index,
                total_charge.astype(jnp.float32), W_wi, b_wi, W_qi, b_qi)


# speedup vs baseline: 3.1247x; 1.1242x over previous
import jax
import jax.numpy as jnp
from jax.experimental import pallas as pl
from jax.experimental.pallas import tpu as pltpu

N = 32768
D = 512
C = 10
S = 16
BN = 4096
GRID = N // BN


def _fused(emb_ref, bi_ref, w_ref, b_ref, qtot_ref, qt_ref,
           hact_s, oh_s, sums_s):
    i = pl.program_id(0)
    h = jax.lax.dot_general(
        w_ref[...], emb_ref[...], (((0,), (1,)), ((), ())),
        preferred_element_type=jnp.float32) + b_ref[...]
    row = jax.lax.broadcasted_iota(jnp.int32, (2 * C, BN), 0)
    hact = jnp.where(row < C, jax.nn.softplus(h), h)
    hact_s[:, pl.ds(i * BN, BN)] = hact

    bi = bi_ref[...]
    oh = (bi == jax.lax.broadcasted_iota(jnp.int32, (S, BN), 0)
          ).astype(jnp.float32)
    oh_s[:, pl.ds(i * BN, BN)] = oh
    part = jax.lax.dot_general(
        oh, hact, (((1,), (1,)), ((), ())),
        preferred_element_type=jnp.float32)

    @pl.when(i == 0)
    def _init():
        sums_s[...] = part

    @pl.when(i != 0)
    def _acc():
        sums_s[...] += part

    @pl.when(i == GRID - 1)
    def _finale():
        sums = sums_s[...]
        wsum = sums[:, :C]
        qsum = sums[:, C:]
        dq = qtot_ref[...] - qsum
        fsys = jnp.where(wsum > 0, dq / jnp.where(wsum > 0, wsum, 1.0), 0.0)
        f = jax.lax.dot_general(
            fsys, oh_s[...], (((0,), (0,)), ((), ())),
            preferred_element_type=jnp.float32)
        hall = hact_s[...]
        qt_ref[...] = hall[C:, :] + hall[:C, :] * f


@jax.jit
def _run(embedding, batch_index, total_charge, W_wi, b_wi, W_qi, b_qi):
    bi_row = batch_index.reshape(1, N)
    w_cat = jnp.concatenate([W_wi, W_qi], axis=1)
    b_cat = jnp.concatenate([b_wi, b_qi]).reshape(2 * C, 1)
    qtot = total_charge.reshape(S, 1)

    q_t = pl.pallas_call(
        _fused,
        grid=(GRID,),
        in_specs=[
            pl.BlockSpec((BN, D), lambda i: (i, 0)),
            pl.BlockSpec((1, BN), lambda i: (0, i)),
            pl.BlockSpec((D, 2 * C), lambda i: (0, 0)),
            pl.BlockSpec((2 * C, 1), lambda i: (0, 0)),
            pl.BlockSpec((S, 1), lambda i: (0, 0)),
        ],
        out_specs=pl.BlockSpec((C, N), lambda i: (0, 0)),
        out_shape=jax.ShapeDtypeStruct((C, N), jnp.float32),
        scratch_shapes=[
            pltpu.VMEM((2 * C, N), jnp.float32),
            pltpu.VMEM((S, N), jnp.float32),
            pltpu.VMEM((S, 2 * C), jnp.float32),
        ],
    )(embedding, bi_row, w_cat, b_cat, qtot)
    return q_t.T


def kernel(embedding, coordinates, batch_index, natoms, total_charge,
           W_wi, b_wi, W_qi, b_qi):
    del coordinates, natoms
    return _run(embedding.astype(jnp.float32), batch_index,
                total_charge.astype(jnp.float32), W_wi, b_wi, W_qi, b_qi)
